# baseline (speedup 1.0000x reference)
import functools

import jax
import jax.numpy as jnp
from jax import lax
from jax.experimental import pallas as pl
from jax.experimental.pallas import tpu as pltpu
from jax.experimental.pallas import tpu_sc as plsc

VOCAB = 100000
EMBED = 128
BATCH = 4096
SEQ = 50

NC = 2
NS = 16
NW = NC * NS
BPW = BATCH // NW
L = 16
NCH = EMBED // L
NBUF = 6
RSTRIDE = 104

_mesh = plsc.VectorSubcoreMesh(core_axis_name="c", subcore_axis_name="s")


def _sc_body(x1f_hbm, x2f_hbm, emb_hbm, wb_hbm, of_hbm, *refs):
  idx1f_v = refs[0]
  idx2f_v = refs[1]
  idxf_v = refs[2]
  rows = list(refs[3:3 + NBUF])
  dots_v, ss1_v, ss2_v, out01_v, wb_v = refs[3 + NBUF:8 + NBUF]
  sems = list(refs[8 + NBUF:8 + 2 * NBUF])

  wid = lax.axis_index("s") * NC + lax.axis_index("c")
  base = wid * BPW

  pltpu.sync_copy(
      x1f_hbm.at[pl.ds(base * SEQ, BPW * SEQ)],
      idx1f_v.at[pl.ds(0, BPW * SEQ)])
  pltpu.sync_copy(
      x2f_hbm.at[pl.ds(base * SEQ, BPW * SEQ)],
      idx2f_v.at[pl.ds(0, BPW * SEQ)])
  pltpu.sync_copy(wb_hbm, wb_v)

  lanes = lax.iota(jnp.int32, L)
  tailmask = lanes < (SEQ - 3 * L)

  def fuse_row(b):
    dst = b * RSTRIDE + lanes
    for c in range(4):
      m = None if c < 3 else tailmask
      v1 = idx1f_v[pl.ds(b * SEQ + c * L, L)]
      plsc.store_scatter(idxf_v, [dst + c * L], v1, mask=m)
      v2 = idx2f_v[pl.ds(b * SEQ + c * L, L)]
      plsc.store_scatter(idxf_v, [dst + SEQ + c * L], v2, mask=m)

  def gather(b, j):
    return pltpu.make_async_copy(
        emb_hbm.at[idxf_v.at[pl.ds(b * RSTRIDE, 2 * SEQ)]], rows[j], sems[j])

  lastmask = lanes == L - 1

  def accumulate(rows_v, b):
    def rbody(r, accs):
      new = []
      for c in range(NCH):
        new.append(accs[c] + rows_v[r, pl.ds(c * L, L)])
      for c in range(NCH):
        new.append(accs[NCH + c] + rows_v[SEQ + r, pl.ds(c * L, L)])
      return tuple(new)

    init = tuple(
        [rows_v[0, pl.ds(c * L, L)] for c in range(NCH)]
        + [rows_v[SEQ, pl.ds(c * L, L)] for c in range(NCH)])
    accs = lax.fori_loop(1, SEQ, rbody, init, unroll=2)
    dotv = accs[0] * accs[NCH]
    ss1v = accs[0] * accs[0]
    ss2v = accs[NCH] * accs[NCH]
    for c in range(1, NCH):
      dotv += accs[c] * accs[NCH + c]
      ss1v += accs[c] * accs[c]
      ss2v += accs[NCH + c] * accs[NCH + c]
    bvec = jnp.full((L,), b, jnp.int32)
    plsc.store_scatter(dots_v, [bvec], jnp.cumsum(dotv), mask=lastmask)
    plsc.store_scatter(ss1_v, [bvec], jnp.cumsum(ss1v), mask=lastmask)
    plsc.store_scatter(ss2_v, [bvec], jnp.cumsum(ss2v), mask=lastmask)

  for j in range(NBUF - 1):
    fuse_row(j)
    gather(j, j).start()

  def group(i, carry):
    b0 = NBUF * i
    for j in range(NBUF):
      b = b0 + j
      gather(b, j).wait()

      @pl.when(b + NBUF - 1 < BPW)
      def _():
        fuse_row(b + NBUF - 1)
        gather(b + NBUF - 1, (j + NBUF - 1) % NBUF).start()

      accumulate(rows[j], b)

    return carry

  n_groups = BPW // NBUF
  lax.fori_loop(0, n_groups, group, 0)
  for b in range(n_groups * NBUF, BPW):
    gather(b, b % NBUF).wait()
    accumulate(rows[b % NBUF], b)

  wv = wb_v[...]
  w0 = wv[0]
  w1 = wv[1]
  b0_ = wv[2]
  b1_ = wv[3]
  inv2 = 1.0 / (SEQ * SEQ)
  for g in range(BPW // L):
    sl = pl.ds(g * L, L)
    dot = dots_v[sl] * inv2
    m1 = jnp.maximum(ss1_v[sl] * inv2, 1e-16)
    m2 = jnp.maximum(ss2_v[sl] * inv2, 1e-16)
    y = m1 * m2
    xi = jnp.int32(0x5F3759DF) - (lax.bitcast_convert_type(y, jnp.int32) >> 1)
    r = lax.bitcast_convert_type(xi, jnp.float32)
    for _ in range(3):
      r = r * (1.5 - 0.5 * y * r * r)
    cos = dot * r
    ids = (jnp.int32(g * L) + lanes) * 2
    plsc.store_scatter(out01_v, [ids], cos * w0 + b0_)
    plsc.store_scatter(out01_v, [ids + 1], cos * w1 + b1_)

  pltpu.sync_copy(out01_v, of_hbm.at[pl.ds(2 * base, 2 * BPW)])


_sc_pool = functools.partial(
    pl.kernel,
    out_type=jax.ShapeDtypeStruct((2 * BATCH,), jnp.float32),
    mesh=_mesh,
    compiler_params=pltpu.CompilerParams(needs_layout_passes=False),
    scratch_types=(
        [pltpu.VMEM((BPW * SEQ + L,), jnp.int32),
         pltpu.VMEM((BPW * SEQ + L,), jnp.int32),
         pltpu.VMEM((BPW * RSTRIDE,), jnp.int32)]
        + [pltpu.VMEM((2 * SEQ, EMBED), jnp.float32) for _ in range(NBUF)]
        + [pltpu.VMEM((BPW,), jnp.float32) for _ in range(3)]
        + [pltpu.VMEM((2 * BPW,), jnp.float32)]
        + [pltpu.VMEM((L,), jnp.float32)]
        + [pltpu.SemaphoreType.DMA for _ in range(NBUF)]
    ),
)(_sc_body)


def kernel(x1, x2, embedding, fc_w, fc_b):
  x1f = x1.astype(jnp.int32).reshape(-1)
  x2f = x2.astype(jnp.int32).reshape(-1)
  wb = jnp.concatenate(
      [fc_w.reshape(-1), fc_b, jnp.zeros((12,), jnp.float32)])
  of = _sc_pool(x1f, x2f, embedding, wb)
  return of.reshape(BATCH, 2)


# speedup vs baseline: 1.0492x; 1.0492x over previous
import functools

import jax
import jax.numpy as jnp
from jax import lax
from jax.experimental import pallas as pl
from jax.experimental.pallas import tpu as pltpu
from jax.experimental.pallas import tpu_sc as plsc

VOCAB = 100000
EMBED = 128
BATCH = 4096
SEQ = 50

NC = 2
NS = 16
NW = NC * NS
BPW = BATCH // NW
L = 16
NCH = EMBED // L
NBUF = 6

_mesh = plsc.VectorSubcoreMesh(core_axis_name="c", subcore_axis_name="s")


def _sc_body(xcat_hbm, emb_hbm, wb_hbm, of_hbm, *refs):
  idx_v = refs[0]
  rows = list(refs[1:1 + NBUF])
  dots_v, ss1_v, ss2_v, out01_v, wb_v = refs[1 + NBUF:6 + NBUF]
  sems = list(refs[6 + NBUF:6 + 2 * NBUF])

  wid = lax.axis_index("s") * NC + lax.axis_index("c")
  base = wid * BPW

  pltpu.sync_copy(xcat_hbm.at[pl.ds(base, BPW)], idx_v)
  pltpu.sync_copy(wb_hbm, wb_v)

  def gather(b, j):
    return pltpu.make_async_copy(emb_hbm.at[idx_v.at[b]], rows[j], sems[j])

  def accumulate(rows_v, b):
    def rbody(r, accs):
      new = []
      for c in range(NCH):
        new.append(accs[c] + rows_v[r, pl.ds(c * L, L)])
      for c in range(NCH):
        new.append(accs[NCH + c] + rows_v[SEQ + r, pl.ds(c * L, L)])
      return tuple(new)

    init = tuple(
        [rows_v[0, pl.ds(c * L, L)] for c in range(NCH)]
        + [rows_v[SEQ, pl.ds(c * L, L)] for c in range(NCH)])
    accs = lax.fori_loop(1, SEQ, rbody, init, unroll=2)
    dotv = accs[0] * accs[NCH]
    ss1v = accs[0] * accs[0]
    ss2v = accs[NCH] * accs[NCH]
    for c in range(1, NCH):
      dotv += accs[c] * accs[NCH + c]
      ss1v += accs[c] * accs[c]
      ss2v += accs[NCH + c] * accs[NCH + c]
    lastmask = lax.iota(jnp.int32, 16) == 15
    bvec = jnp.full((L,), b, jnp.int32)
    plsc.store_scatter(dots_v, [bvec], jnp.cumsum(dotv), mask=lastmask)
    plsc.store_scatter(ss1_v, [bvec], jnp.cumsum(ss1v), mask=lastmask)
    plsc.store_scatter(ss2_v, [bvec], jnp.cumsum(ss2v), mask=lastmask)

  for j in range(NBUF - 1):
    gather(j, j).start()

  def group(i, carry):
    b0 = NBUF * i
    for j in range(NBUF):
      b = b0 + j
      gather(b, j).wait()

      @pl.when(b + NBUF - 1 < BPW)
      def _():
        gather(b + NBUF - 1, (j + NBUF - 1) % NBUF).start()

      accumulate(rows[j], b)

    return carry

  n_groups = BPW // NBUF
  lax.fori_loop(0, n_groups, group, 0)
  for b in range(n_groups * NBUF, BPW):
    gather(b, b % NBUF).wait()
    accumulate(rows[b % NBUF], b)

  wv = wb_v[...]
  w0 = wv[0]
  w1 = wv[1]
  b0_ = wv[2]
  b1_ = wv[3]
  inv2 = 1.0 / (SEQ * SEQ)
  for g in range(BPW // L):
    sl = pl.ds(g * L, L)
    dot = dots_v[sl] * inv2
    m1 = jnp.maximum(ss1_v[sl] * inv2, 1e-16)
    m2 = jnp.maximum(ss2_v[sl] * inv2, 1e-16)
    y = m1 * m2
    xi = jnp.int32(0x5F3759DF) - (lax.bitcast_convert_type(y, jnp.int32) >> 1)
    r = lax.bitcast_convert_type(xi, jnp.float32)
    for _ in range(3):
      r = r * (1.5 - 0.5 * y * r * r)
    cos = dot * r
    ids = (jnp.int32(g * L) + lax.iota(jnp.int32, L)) * 2
    plsc.store_scatter(out01_v, [ids], cos * w0 + b0_)
    plsc.store_scatter(out01_v, [ids + 1], cos * w1 + b1_)

  pltpu.sync_copy(out01_v, of_hbm.at[pl.ds(2 * base, 2 * BPW)])


_sc_pool = functools.partial(
    pl.kernel,
    out_type=jax.ShapeDtypeStruct((2 * BATCH,), jnp.float32),
    mesh=_mesh,
    compiler_params=pltpu.CompilerParams(needs_layout_passes=False),
    scratch_types=(
        [pltpu.VMEM((BPW, 2 * SEQ), jnp.int32)]
        + [pltpu.VMEM((2 * SEQ, EMBED), jnp.float32) for _ in range(NBUF)]
        + [pltpu.VMEM((BPW,), jnp.float32) for _ in range(3)]
        + [pltpu.VMEM((2 * BPW,), jnp.float32)]
        + [pltpu.VMEM((L,), jnp.float32)]
        + [pltpu.SemaphoreType.DMA for _ in range(NBUF)]
    ),
)(_sc_body)


def kernel(x1, x2, embedding, fc_w, fc_b):
  xcat = jnp.concatenate(
      [x1.astype(jnp.int32), x2.astype(jnp.int32)], axis=1)
  wb = jnp.concatenate(
      [fc_w.reshape(-1), fc_b, jnp.zeros((12,), jnp.float32)])
  of = _sc_pool(xcat, embedding, wb)
  return of.reshape(BATCH, 2)


# speedup vs baseline: 1.0773x; 1.0268x over previous
import functools

import jax
import jax.numpy as jnp
from jax import lax
from jax.experimental import pallas as pl
from jax.experimental.pallas import tpu as pltpu
from jax.experimental.pallas import tpu_sc as plsc

VOCAB = 100000
EMBED = 128
BATCH = 4096
SEQ = 50

NC = 2
NS = 16
NW = NC * NS
BPW = BATCH // NW
L = 16
NCH = EMBED // L
NBUF = 6

_mesh = plsc.VectorSubcoreMesh(core_axis_name="c", subcore_axis_name="s")


def _sc_body(xcat_hbm, emb_hbm, wb_hbm, o0_hbm, o1_hbm, *refs):
  idx_v = refs[0]
  rows = list(refs[1:1 + NBUF])
  dots_v, ss1_v, ss2_v, out0_v, out1_v, wb_v = refs[1 + NBUF:7 + NBUF]
  sems = list(refs[7 + NBUF:7 + 2 * NBUF])

  wid = lax.axis_index("s") * NC + lax.axis_index("c")
  base = wid * BPW

  pltpu.sync_copy(xcat_hbm.at[pl.ds(base, BPW)], idx_v)
  pltpu.sync_copy(wb_hbm, wb_v)

  def gather(b, j):
    return pltpu.make_async_copy(emb_hbm.at[idx_v.at[b]], rows[j], sems[j])

  def accumulate(rows_v, b):
    def rbody(r, accs):
      new = []
      for c in range(NCH):
        new.append(accs[c] + rows_v[r, pl.ds(c * L, L)])
      for c in range(NCH):
        new.append(accs[NCH + c] + rows_v[SEQ + r, pl.ds(c * L, L)])
      return tuple(new)

    init = tuple(
        [rows_v[0, pl.ds(c * L, L)] for c in range(NCH)]
        + [rows_v[SEQ, pl.ds(c * L, L)] for c in range(NCH)])
    accs = lax.fori_loop(1, SEQ, rbody, init, unroll=2)
    dotv = accs[0] * accs[NCH]
    ss1v = accs[0] * accs[0]
    ss2v = accs[NCH] * accs[NCH]
    for c in range(1, NCH):
      dotv += accs[c] * accs[NCH + c]
      ss1v += accs[c] * accs[c]
      ss2v += accs[NCH + c] * accs[NCH + c]
    lastmask = lax.iota(jnp.int32, 16) == 15
    bvec = jnp.full((L,), b, jnp.int32)
    plsc.store_scatter(dots_v, [bvec], jnp.cumsum(dotv), mask=lastmask)
    plsc.store_scatter(ss1_v, [bvec], jnp.cumsum(ss1v), mask=lastmask)
    plsc.store_scatter(ss2_v, [bvec], jnp.cumsum(ss2v), mask=lastmask)

  for j in range(NBUF - 1):
    gather(j, j).start()

  def group(i, carry):
    b0 = NBUF * i
    for j in range(NBUF):
      b = b0 + j
      gather(b, j).wait()

      @pl.when(b + NBUF - 1 < BPW)
      def _():
        gather(b + NBUF - 1, (j + NBUF - 1) % NBUF).start()

      accumulate(rows[j], b)

    return carry

  n_groups = BPW // NBUF
  lax.fori_loop(0, n_groups, group, 0)
  for b in range(n_groups * NBUF, BPW):
    gather(b, b % NBUF).wait()
    accumulate(rows[b % NBUF], b)

  wv = wb_v[...]
  w0 = wv[0]
  w1 = wv[1]
  b0_ = wv[2]
  b1_ = wv[3]
  inv2 = 1.0 / (SEQ * SEQ)
  for g in range(BPW // L):
    sl = pl.ds(g * L, L)
    dot = dots_v[sl] * inv2
    m1 = jnp.maximum(ss1_v[sl] * inv2, 1e-16)
    m2 = jnp.maximum(ss2_v[sl] * inv2, 1e-16)
    y = m1 * m2
    xi = jnp.int32(0x5F3759DF) - (lax.bitcast_convert_type(y, jnp.int32) >> 1)
    r = lax.bitcast_convert_type(xi, jnp.float32)
    for _ in range(3):
      r = r * (1.5 - 0.5 * y * r * r)
    cos = dot * r
    out0_v[sl] = cos * w0 + b0_
    out1_v[sl] = cos * w1 + b1_

  pltpu.sync_copy(out0_v, o0_hbm.at[pl.ds(base, BPW)])
  pltpu.sync_copy(out1_v, o1_hbm.at[pl.ds(base, BPW)])


_sc_pool = functools.partial(
    pl.kernel,
    out_type=(
        jax.ShapeDtypeStruct((BATCH,), jnp.float32),
        jax.ShapeDtypeStruct((BATCH,), jnp.float32),
    ),
    mesh=_mesh,
    compiler_params=pltpu.CompilerParams(needs_layout_passes=False),
    scratch_types=(
        [pltpu.VMEM((BPW, 2 * SEQ), jnp.int32)]
        + [pltpu.VMEM((2 * SEQ, EMBED), jnp.float32) for _ in range(NBUF)]
        + [pltpu.VMEM((BPW,), jnp.float32) for _ in range(5)]
        + [pltpu.VMEM((L,), jnp.float32)]
        + [pltpu.SemaphoreType.DMA for _ in range(NBUF)]
    ),
)(_sc_body)


def kernel(x1, x2, embedding, fc_w, fc_b):
  xcat = jnp.concatenate(
      [x1.astype(jnp.int32), x2.astype(jnp.int32)], axis=1)
  wb = jnp.concatenate(
      [fc_w.reshape(-1), fc_b, jnp.zeros((12,), jnp.float32)])
  o0, o1 = _sc_pool(xcat, embedding, wb)
  return jnp.stack([o0, o1], axis=1)
